# baseline (speedup 1.0000x reference)
import jax
import jax.numpy as jnp
from jax.experimental import pallas as pl

T, B, D = 197, 32, 384
E = 8
K = 2
CAP = 49
CP = 56
H = D * 4
GH = D // 4
BA = 4
NBA = B // BA


def _route_one(xb, gw1, gb1, gw2, gb2, s_lt_t):
    g = jax.lax.dot_general(gw1, xb, (((1,), (1,)), ((), ())),
                            preferred_element_type=jnp.float32)
    g = jnp.maximum(g + gb1, 0.0)
    logits = jax.lax.dot_general(gw2, g, (((1,), (0,)), ((), ())),
                                 preferred_element_type=jnp.float32)
    logits = logits + gb2
    m = jnp.max(logits, axis=0, keepdims=True)
    p = jnp.exp(logits - m)
    gating = p / jnp.sum(p, axis=0, keepdims=True)

    ge = gating[:, None, :]
    gf = gating[None, :, :]
    f_lt_e = (jax.lax.broadcasted_iota(jnp.int32, (E, E, T), 1)
              < jax.lax.broadcasted_iota(jnp.int32, (E, E, T), 0))
    rank_e = (jnp.sum((gf > ge).astype(jnp.int32), axis=1)
              + jnp.sum(((gf == ge) & f_lt_e).astype(jnp.int32), axis=1))
    chosen = gating * (rank_e < K).astype(jnp.float32)

    ct = jnp.transpose(chosen)
    cols = []
    for e in range(E):
        vs = jnp.broadcast_to(chosen[e:e + 1, :], (T, T))
        vt = jnp.broadcast_to(ct[:, e:e + 1], (T, T))
        ahead = (vs > vt) | ((vs == vt) & s_lt_t)
        cols.append(jnp.sum(ahead.astype(jnp.int32), axis=1, keepdims=True))
    rank_t = jnp.concatenate(cols, axis=1)
    rank_c = jnp.transpose(rank_t)
    sel = rank_c < CAP
    slot = jnp.where(sel, rank_c, 1000)
    fw = chosen * sel.astype(jnp.float32)

    c_iota = jax.lax.broadcasted_iota(jnp.int32, (E, CP, T), 1)
    p8 = (slot[:, None, :] == c_iota).astype(jnp.bfloat16)
    xg = jax.lax.dot_general(p8.reshape(E * CP, T), xb.astype(jnp.bfloat16),
                             (((1,), (0,)), ((), ())),
                             preferred_element_type=jnp.float32)
    return slot, fw, xg.astype(jnp.bfloat16)


def _routing_kernel(x_ref, gw1_ref, gb1_ref, gw2_ref, gb2_ref,
                    slot_ref, fw_ref, xg_ref):
    s_lt_t = (jax.lax.broadcasted_iota(jnp.int32, (T, T), 1)
              < jax.lax.broadcasted_iota(jnp.int32, (T, T), 0))
    gw1 = gw1_ref[...]
    gb1 = gb1_ref[...]
    gw2 = gw2_ref[...]
    gb2 = gb2_ref[...]
    for bl in range(BA):
        xb = x_ref[:, bl * D:(bl + 1) * D]
        slot, fw, xg = _route_one(xb, gw1, gb1, gw2, gb2, s_lt_t)
        slot_ref[bl * E:(bl + 1) * E] = slot.reshape(E, 1, T)
        fw_ref[bl * E:(bl + 1) * E] = fw.reshape(E, 1, T)
        xg_ref[bl] = xg.reshape(E, CP, D)


def _ffn_kernel(xg_ref, wfc_ref, bfc_ref, wpj_ref, bpj_ref, y_ref):
    xg = xg_ref[...].reshape(B * CP, D)
    h = jax.lax.dot_general(xg, wfc_ref[0].astype(jnp.bfloat16),
                            (((1,), (1,)), ((), ())),
                            preferred_element_type=jnp.float32)
    h = jnp.maximum(h + bfc_ref[0], 0.0)
    y = jax.lax.dot_general(h.astype(jnp.bfloat16),
                            wpj_ref[0].astype(jnp.bfloat16),
                            (((1,), (1,)), ((), ())),
                            preferred_element_type=jnp.float32)
    y = y + bpj_ref[0]
    y_ref[...] = y.astype(jnp.bfloat16).reshape(B, 1, CP, D)


def _combine_kernel(y_ref, slot_ref, fw_ref, x_ref, out_ref):
    c_iota = jax.lax.broadcasted_iota(jnp.int32, (E, CP, T), 1)
    for bl in range(BA):
        slot = slot_ref[bl * E:(bl + 1) * E]
        fw = fw_ref[bl * E:(bl + 1) * E]
        w2t = jnp.where(slot == c_iota, fw, 0.0).astype(jnp.bfloat16)
        yb = y_ref[bl].reshape(E * CP, D)
        acc = jax.lax.dot_general(w2t.reshape(E * CP, T), yb,
                                  (((0,), (0,)), ((), ())),
                                  preferred_element_type=jnp.float32)
        out_ref[:, bl * D:(bl + 1) * D] = acc - x_ref[:, bl * D:(bl + 1) * D]


@jax.jit
def kernel(x, gW1, gb1, gW2, gb2, Wfc, bfc, Wproj, bproj):
    x2 = x.reshape(T, B * D)
    gb1c = gb1.reshape(GH, 1)
    gb2c = gb2.reshape(E, 1)
    bfc3 = bfc.reshape(E, 1, H)
    bpj3 = bproj.reshape(E, 1, D)

    slot, fw, xg = pl.pallas_call(
        _routing_kernel,
        grid=(NBA,),
        in_specs=[
            pl.BlockSpec((T, BA * D), lambda b: (0, b)),
            pl.BlockSpec((GH, D), lambda b: (0, 0)),
            pl.BlockSpec((GH, 1), lambda b: (0, 0)),
            pl.BlockSpec((E, GH), lambda b: (0, 0)),
            pl.BlockSpec((E, 1), lambda b: (0, 0)),
        ],
        out_specs=[
            pl.BlockSpec((BA * E, 1, T), lambda b: (b, 0, 0)),
            pl.BlockSpec((BA * E, 1, T), lambda b: (b, 0, 0)),
            pl.BlockSpec((BA, E, CP, D), lambda b: (b, 0, 0, 0)),
        ],
        out_shape=[
            jax.ShapeDtypeStruct((B * E, 1, T), jnp.int32),
            jax.ShapeDtypeStruct((B * E, 1, T), jnp.float32),
            jax.ShapeDtypeStruct((B, E, CP, D), jnp.bfloat16),
        ],
    )(x2, gW1, gb1c, gW2, gb2c)

    y = pl.pallas_call(
        _ffn_kernel,
        grid=(E,),
        in_specs=[
            pl.BlockSpec((B, 1, CP, D), lambda e: (0, e, 0, 0)),
            pl.BlockSpec((1, H, D), lambda e: (e, 0, 0)),
            pl.BlockSpec((1, 1, H), lambda e: (e, 0, 0)),
            pl.BlockSpec((1, D, H), lambda e: (e, 0, 0)),
            pl.BlockSpec((1, 1, D), lambda e: (e, 0, 0)),
        ],
        out_specs=pl.BlockSpec((B, 1, CP, D), lambda e: (0, e, 0, 0)),
        out_shape=jax.ShapeDtypeStruct((B, E, CP, D), jnp.bfloat16),
    )(xg, Wfc, bfc3, Wproj, bpj3)

    out2 = pl.pallas_call(
        _combine_kernel,
        grid=(NBA,),
        in_specs=[
            pl.BlockSpec((BA, E, CP, D), lambda b: (b, 0, 0, 0)),
            pl.BlockSpec((BA * E, 1, T), lambda b: (b, 0, 0)),
            pl.BlockSpec((BA * E, 1, T), lambda b: (b, 0, 0)),
            pl.BlockSpec((T, BA * D), lambda b: (0, b)),
        ],
        out_specs=pl.BlockSpec((T, BA * D), lambda b: (0, b)),
        out_shape=jax.ShapeDtypeStruct((T, B * D), jnp.float32),
    )(y, slot, fw, x2)

    return out2.reshape(T, B, D)


# speedup vs baseline: 1.0775x; 1.0775x over previous
import jax
import jax.numpy as jnp
from jax.experimental import pallas as pl

T, B, D = 197, 32, 384
E = 8
K = 2
CAP = 49
CP = 56
H = D * 4
GH = D // 4
BA = 4
NBA = B // BA


def _route_one(xb, gw1, gb1, gw2, gb2, s_lt_t):
    g = jax.lax.dot_general(gw1, xb, (((1,), (1,)), ((), ())),
                            preferred_element_type=jnp.float32)
    g = jnp.maximum(g + gb1, 0.0)
    logits = jax.lax.dot_general(gw2, g, (((1,), (0,)), ((), ())),
                                 preferred_element_type=jnp.float32)
    logits = logits + gb2
    m = jnp.max(logits, axis=0, keepdims=True)
    p = jnp.exp(logits - m)
    gating = p / jnp.sum(p, axis=0, keepdims=True)

    ge = gating[:, None, :]
    gf = gating[None, :, :]
    f_lt_e = (jax.lax.broadcasted_iota(jnp.int32, (E, E, T), 1)
              < jax.lax.broadcasted_iota(jnp.int32, (E, E, T), 0))
    rank_e = (jnp.sum((gf > ge).astype(jnp.int32), axis=1)
              + jnp.sum(((gf == ge) & f_lt_e).astype(jnp.int32), axis=1))
    chosen = gating * (rank_e < K).astype(jnp.float32)

    ct = jnp.transpose(chosen)
    ones_t = jnp.ones((T, 1), jnp.float32)
    cols = []
    for e in range(E):
        vs = jnp.broadcast_to(chosen[e:e + 1, :], (T, T))
        vt = jnp.broadcast_to(ct[:, e:e + 1], (T, T))
        ahead = ((vs > vt) | ((vs == vt) & s_lt_t)).astype(jnp.float32)
        cols.append(jax.lax.dot_general(ahead, ones_t, (((1,), (0,)), ((), ())),
                                        preferred_element_type=jnp.float32))
    rank_t = jnp.concatenate(cols, axis=1)
    rank_c = jnp.transpose(rank_t)
    sel = rank_c < float(CAP)
    slot = jnp.where(sel, rank_c, 1000.0).astype(jnp.int32)
    fw = chosen * sel.astype(jnp.float32)

    c_iota = jax.lax.broadcasted_iota(jnp.int32, (E, CP, T), 1)
    p8 = (slot[:, None, :] == c_iota).astype(jnp.bfloat16)
    xg = jax.lax.dot_general(p8.reshape(E * CP, T), xb.astype(jnp.bfloat16),
                             (((1,), (0,)), ((), ())),
                             preferred_element_type=jnp.float32)
    return slot, fw, xg.astype(jnp.bfloat16)


def _routing_kernel(x_ref, gw1_ref, gb1_ref, gw2_ref, gb2_ref,
                    slot_ref, fw_ref, xg_ref):
    s_lt_t = (jax.lax.broadcasted_iota(jnp.int32, (T, T), 1)
              < jax.lax.broadcasted_iota(jnp.int32, (T, T), 0))
    gw1 = gw1_ref[...]
    gb1 = gb1_ref[...]
    gw2 = gw2_ref[...]
    gb2 = gb2_ref[...]
    for bl in range(BA):
        xb = x_ref[:, bl * D:(bl + 1) * D]
        slot, fw, xg = _route_one(xb, gw1, gb1, gw2, gb2, s_lt_t)
        slot_ref[bl * E:(bl + 1) * E] = slot.reshape(E, 1, T)
        fw_ref[bl * E:(bl + 1) * E] = fw.reshape(E, 1, T)
        xg_ref[bl] = xg.reshape(E, CP, D)


def _ffn_kernel(xg_ref, wfc_ref, bfc_ref, wpj_ref, bpj_ref, y_ref):
    xg = xg_ref[...].reshape(B * CP, D)
    h = jax.lax.dot_general(xg, wfc_ref[0].astype(jnp.bfloat16),
                            (((1,), (1,)), ((), ())),
                            preferred_element_type=jnp.float32)
    h = jnp.maximum(h + bfc_ref[0], 0.0)
    y = jax.lax.dot_general(h.astype(jnp.bfloat16),
                            wpj_ref[0].astype(jnp.bfloat16),
                            (((1,), (1,)), ((), ())),
                            preferred_element_type=jnp.float32)
    y = y + bpj_ref[0]
    y_ref[...] = y.astype(jnp.bfloat16).reshape(B, 1, CP, D)


def _combine_kernel(y_ref, slot_ref, fw_ref, x_ref, out_ref):
    c_iota = jax.lax.broadcasted_iota(jnp.int32, (E, CP, T), 1)
    for bl in range(BA):
        slot = slot_ref[bl * E:(bl + 1) * E]
        fw = fw_ref[bl * E:(bl + 1) * E]
        w2t = jnp.where(slot == c_iota, fw, 0.0).astype(jnp.bfloat16)
        yb = y_ref[bl].reshape(E * CP, D)
        acc = jax.lax.dot_general(w2t.reshape(E * CP, T), yb,
                                  (((0,), (0,)), ((), ())),
                                  preferred_element_type=jnp.float32)
        out_ref[:, bl * D:(bl + 1) * D] = acc - x_ref[:, bl * D:(bl + 1) * D]


@jax.jit
def kernel(x, gW1, gb1, gW2, gb2, Wfc, bfc, Wproj, bproj):
    x2 = x.reshape(T, B * D)
    gb1c = gb1.reshape(GH, 1)
    gb2c = gb2.reshape(E, 1)
    bfc3 = bfc.reshape(E, 1, H)
    bpj3 = bproj.reshape(E, 1, D)

    slot, fw, xg = pl.pallas_call(
        _routing_kernel,
        grid=(NBA,),
        in_specs=[
            pl.BlockSpec((T, BA * D), lambda b: (0, b)),
            pl.BlockSpec((GH, D), lambda b: (0, 0)),
            pl.BlockSpec((GH, 1), lambda b: (0, 0)),
            pl.BlockSpec((E, GH), lambda b: (0, 0)),
            pl.BlockSpec((E, 1), lambda b: (0, 0)),
        ],
        out_specs=[
            pl.BlockSpec((BA * E, 1, T), lambda b: (b, 0, 0)),
            pl.BlockSpec((BA * E, 1, T), lambda b: (b, 0, 0)),
            pl.BlockSpec((BA, E, CP, D), lambda b: (b, 0, 0, 0)),
        ],
        out_shape=[
            jax.ShapeDtypeStruct((B * E, 1, T), jnp.int32),
            jax.ShapeDtypeStruct((B * E, 1, T), jnp.float32),
            jax.ShapeDtypeStruct((B, E, CP, D), jnp.bfloat16),
        ],
    )(x2, gW1, gb1c, gW2, gb2c)

    y = pl.pallas_call(
        _ffn_kernel,
        grid=(E,),
        in_specs=[
            pl.BlockSpec((B, 1, CP, D), lambda e: (0, e, 0, 0)),
            pl.BlockSpec((1, H, D), lambda e: (e, 0, 0)),
            pl.BlockSpec((1, 1, H), lambda e: (e, 0, 0)),
            pl.BlockSpec((1, D, H), lambda e: (e, 0, 0)),
            pl.BlockSpec((1, 1, D), lambda e: (e, 0, 0)),
        ],
        out_specs=pl.BlockSpec((B, 1, CP, D), lambda e: (0, e, 0, 0)),
        out_shape=jax.ShapeDtypeStruct((B, E, CP, D), jnp.bfloat16),
    )(xg, Wfc, bfc3, Wproj, bpj3)

    out2 = pl.pallas_call(
        _combine_kernel,
        grid=(NBA,),
        in_specs=[
            pl.BlockSpec((BA, E, CP, D), lambda b: (b, 0, 0, 0)),
            pl.BlockSpec((BA * E, 1, T), lambda b: (b, 0, 0)),
            pl.BlockSpec((BA * E, 1, T), lambda b: (b, 0, 0)),
            pl.BlockSpec((T, BA * D), lambda b: (0, b)),
        ],
        out_specs=pl.BlockSpec((T, BA * D), lambda b: (0, b)),
        out_shape=jax.ShapeDtypeStruct((T, B * D), jnp.float32),
    )(y, slot, fw, x2)

    return out2.reshape(T, B, D)
